# Initial kernel scaffold; baseline (speedup 1.0000x reference)
import jax
import jax.numpy as jnp
from jax.experimental import pallas as pl


def kernel(x, edge_index, batch, edge_attr, W1, b1, W2, b2, Wlin, blin):
    raise NotImplementedError("write your pallas kernel here")


# speedup vs baseline: 13.3643x; 13.3643x over previous
import functools

import jax
import jax.numpy as jnp
from jax import lax
from jax.experimental import pallas as pl
from jax.experimental.pallas import tpu as pltpu
from jax.experimental.pallas import tpu_sc as plsc

N = 10000
E = 320000
D = 128

NC = 2
NS = 16
NW = NC * NS
EPW = E // NW
CH = 80
NCHUNK = EPW // CH
RPT = 624
RREM = N - RPT * NS

_MESH = plsc.VectorSubcoreMesh(core_axis_name="c", subcore_axis_name="s")


@functools.partial(
    pl.kernel,
    mesh=_MESH,
    out_type=jax.ShapeDtypeStruct((NC, N), jnp.float32),
    scratch_types=[
        pltpu.VMEM((CH,), jnp.int32),
        pltpu.VMEM((CH,), jnp.float32),
        pltpu.VMEM_SHARED((N,), jnp.float32),
    ],
)
def _deg_sc(edge_hbm, ones_hbm, out_hbm, idx_v, ones_v, acc):
    c = lax.axis_index("c")
    s = lax.axis_index("s")

    @pl.when(s == 0)
    def _():
        pltpu.sync_copy(ones_hbm, acc)

    pltpu.sync_copy(ones_hbm.at[pl.ds(0, CH)], ones_v)
    plsc.subcore_barrier()

    base = (c * NS + s) * EPW

    def body(j, carry):
        pltpu.sync_copy(edge_hbm.at[pl.ds(E + base + j * CH, CH)], idx_v)
        pltpu.sync_copy(ones_v, acc.at[idx_v], add=True)
        return carry

    lax.fori_loop(0, NCHUNK, body, 0)
    plsc.subcore_barrier()

    @pl.when(s == 0)
    def _():
        pltpu.sync_copy(acc, out_hbm.at[c])


@functools.partial(
    pl.kernel,
    mesh=_MESH,
    out_type=jax.ShapeDtypeStruct((NC, N, D), jnp.float32),
    scratch_types=[
        pltpu.VMEM((CH,), jnp.int32),
        pltpu.VMEM((CH,), jnp.int32),
        pltpu.VMEM((CH, D), jnp.float32),
        pltpu.VMEM_SHARED((N, D), jnp.float32),
        pltpu.SemaphoreType.DMA,
    ],
)
def _edge_sc(hs_hbm, edge_hbm, out_hbm, sidx, didx, rows, acc, sem):
    c = lax.axis_index("c")
    s = lax.axis_index("s")

    r0 = s * RPT
    pltpu.sync_copy(hs_hbm.at[pl.ds(r0, RPT)], acc.at[pl.ds(r0, RPT)])

    @pl.when(s == NS - 1)
    def _():
        pltpu.sync_copy(hs_hbm.at[pl.ds(RPT * NS, RREM)],
                        acc.at[pl.ds(RPT * NS, RREM)])

    plsc.subcore_barrier()

    base = (c * NS + s) * EPW

    def body(j, carry):
        e0 = base + j * CH
        pltpu.sync_copy(edge_hbm.at[pl.ds(e0, CH)], sidx)
        pltpu.sync_copy(edge_hbm.at[pl.ds(E + e0, CH)], didx)
        pltpu.async_copy(hs_hbm.at[sidx], rows, sem).wait()
        pltpu.sync_copy(rows, acc.at[didx], add=True)
        return carry

    lax.fori_loop(0, NCHUNK, body, 0)
    plsc.subcore_barrier()
    pltpu.sync_copy(acc.at[pl.ds(r0, RPT)], out_hbm.at[c, pl.ds(r0, RPT)])

    @pl.when(s == NS - 1)
    def _():
        pltpu.sync_copy(acc.at[pl.ds(RPT * NS, RREM)],
                        out_hbm.at[c, pl.ds(RPT * NS, RREM)])


def _dinv(degp_t_ref):
    deg = degp_t_ref[:, 0:1] + degp_t_ref[:, 1:2] - 1.0
    return lax.rsqrt(deg)


def _tc1_body(degp_t_ref, x_ref, w_ref, hs_ref):
    dinv = _dinv(degp_t_ref)
    hs_ref[...] = jnp.dot(x_ref[...], w_ref[...],
                          preferred_element_type=jnp.float32) * dinv


_tc1 = pl.pallas_call(
    _tc1_body,
    out_shape=jax.ShapeDtypeStruct((N, D), jnp.float32),
)


def _tc2_body(degp_t_ref, p_ref, hs_ref, b_ref, w_ref, out_ref):
    dinv = _dinv(degp_t_ref)
    agg = p_ref[0] + p_ref[1] - hs_ref[...]
    z = jnp.maximum(agg * dinv + b_ref[...], 0.0)
    out_ref[...] = jnp.dot(z, w_ref[...],
                           preferred_element_type=jnp.float32) * dinv


_tc2 = pl.pallas_call(
    _tc2_body,
    out_shape=jax.ShapeDtypeStruct((N, D), jnp.float32),
)


def _tc3_body(degp_t_ref, p_ref, hs_ref, b_ref, wlin_ref, blin_ref, out_ref):
    dinv = _dinv(degp_t_ref)
    agg = p_ref[0] + p_ref[1] - hs_ref[...]
    z = jnp.maximum(agg * dinv + b_ref[...], 0.0)
    out_ref[...] = jnp.dot(z, wlin_ref[...],
                           preferred_element_type=jnp.float32) + blin_ref[...]


_tc3 = pl.pallas_call(
    _tc3_body,
    out_shape=jax.ShapeDtypeStruct((N, 1), jnp.float32),
)


def kernel(x, edge_index, batch, edge_attr, W1, b1, W2, b2, Wlin, blin):
    del batch, edge_attr
    edges = edge_index.astype(jnp.int32).reshape(2 * E)
    ones = jnp.ones((N,), jnp.float32)

    degp = _deg_sc(edges, ones)
    degp_t = degp.T

    hs1 = _tc1(degp_t, x, W1)
    p1 = _edge_sc(hs1, edges)
    hs2 = _tc2(degp_t, p1, hs1, b1.reshape(1, D), W2)
    p2 = _edge_sc(hs2, edges)
    out = _tc3(degp_t, p2, hs2, b2.reshape(1, D), Wlin, blin.reshape(1, 1))
    return out


# speedup vs baseline: 25.9956x; 1.9451x over previous
import functools

import jax
import jax.numpy as jnp
from jax import lax
from jax.experimental import pallas as pl
from jax.experimental.pallas import tpu as pltpu
from jax.experimental.pallas import tpu_sc as plsc

N = 10000
E = 320000
D = 128

NC = 2
NS = 16
NW = NC * NS
EPW = E // NW
CH = 80
NCHUNK = EPW // CH
RPT = 624
RREM = N - RPT * NS

_MESH = plsc.VectorSubcoreMesh(core_axis_name="c", subcore_axis_name="s")


NB = 2
NWAVE = NCHUNK // NB
NTAIL = NCHUNK - NWAVE * NB


@functools.partial(
    pl.kernel,
    mesh=_MESH,
    out_type=jax.ShapeDtypeStruct((NC, N), jnp.float32),
    scratch_types=[
        pltpu.VMEM((NCHUNK, CH), jnp.int32),
        pltpu.VMEM((CH,), jnp.float32),
        pltpu.VMEM_SHARED((N,), jnp.float32),
        [pltpu.SemaphoreType.DMA] * NB,
    ],
)
def _deg_sc(edge_hbm, ones_hbm, out_hbm, didx, ones_v, acc, sems):
    c = lax.axis_index("c")
    s = lax.axis_index("s")

    @pl.when(s == 0)
    def _():
        pltpu.sync_copy(ones_hbm, acc)

    pltpu.sync_copy(ones_hbm.at[pl.ds(0, CH)], ones_v)
    w = c * NS + s
    pltpu.sync_copy(edge_hbm.at[1, w], didx)
    plsc.subcore_barrier()

    def body(j, carry):
        ds = [
            pltpu.async_copy(ones_v, acc.at[didx.at[j * NB + b]], sems[b],
                             add=True)
            for b in range(NB)
        ]
        for d in ds:
            d.wait()
        return carry

    lax.fori_loop(0, NWAVE, body, 0)
    for t in range(NTAIL):
        pltpu.sync_copy(ones_v, acc.at[didx.at[NWAVE * NB + t]], add=True)
    plsc.subcore_barrier()

    @pl.when(s == 0)
    def _():
        pltpu.sync_copy(acc, out_hbm.at[c])


@functools.partial(
    pl.kernel,
    mesh=_MESH,
    out_type=jax.ShapeDtypeStruct((NC, N, D), jnp.float32),
    scratch_types=[
        pltpu.VMEM((2, CH), jnp.int32),
        pltpu.VMEM((2, CH), jnp.int32),
        pltpu.VMEM((2, CH, D), jnp.float32),
        pltpu.VMEM_SHARED((N, D), jnp.float32),
        pltpu.SemaphoreType.DMA((2,)),
        pltpu.SemaphoreType.DMA((2,)),
        pltpu.SemaphoreType.DMA((2,)),
    ],
)
def _edge_sc(hs_hbm, edge_hbm, eflat_hbm, out_hbm, sidx, didx, rows, acc,
             isem, gsem, ssem):
    c = lax.axis_index("c")
    s = lax.axis_index("s")

    r0 = s * RPT
    pltpu.sync_copy(hs_hbm.at[pl.ds(r0, RPT)], acc.at[pl.ds(r0, RPT)])

    @pl.when(s == NS - 1)
    def _():
        pltpu.sync_copy(hs_hbm.at[pl.ds(RPT * NS, RREM)],
                        acc.at[pl.ds(RPT * NS, RREM)])

    w = c * NS + s
    sb = w * EPW
    db = E + w * EPW
    plsc.subcore_barrier()

    def iload(t, j):
        pltpu.async_copy(eflat_hbm.at[pl.ds(sb + j * CH, CH)], sidx.at[t],
                         isem.at[t])
        pltpu.async_copy(eflat_hbm.at[pl.ds(db + j * CH, CH)], didx.at[t],
                         isem.at[t])
        pltpu.make_async_copy(eflat_hbm.at[pl.ds(0, CH)], sidx.at[t],
                              isem.at[t]).wait()
        pltpu.make_async_copy(eflat_hbm.at[pl.ds(0, CH)], didx.at[t],
                              isem.at[t]).wait()

    def gather(t):
        pltpu.async_copy(hs_hbm.at[sidx.at[t]], rows.at[t], gsem.at[t])

    def gwait(t):
        pltpu.make_async_copy(hs_hbm.at[pl.ds(0, CH)], rows.at[t],
                              gsem.at[t]).wait()

    def scatter(t):
        pltpu.async_copy(rows.at[t], acc.at[didx.at[t]], ssem.at[t], add=True)

    def swait(t):
        pltpu.make_async_copy(hs_hbm.at[pl.ds(0, CH)], rows.at[t],
                              ssem.at[t]).wait()

    K = NCHUNK // 2
    iload(0, 0)
    gather(0)

    def body(k, carry):
        gwait(0)
        scatter(0)
        @pl.when(k > 0)
        def _():
            swait(1)

        iload(1, 2 * k + 1)
        gather(1)
        swait(0)
        iload(0, 2 * k + 2)
        gather(0)
        gwait(1)
        scatter(1)
        return carry

    lax.fori_loop(0, K, body, 0)
    swait(1)
    gwait(0)
    scatter(0)
    swait(0)
    plsc.subcore_barrier()
    pltpu.sync_copy(acc.at[pl.ds(r0, RPT)], out_hbm.at[c, pl.ds(r0, RPT)])

    @pl.when(s == NS - 1)
    def _():
        pltpu.sync_copy(acc.at[pl.ds(RPT * NS, RREM)],
                        out_hbm.at[c, pl.ds(RPT * NS, RREM)])


def _dinv(degp_t_ref):
    deg = degp_t_ref[:, 0:1] + degp_t_ref[:, 1:2] - 1.0
    return lax.rsqrt(deg)


def _tc1_body(degp_t_ref, x_ref, w_ref, hs_ref):
    dinv = _dinv(degp_t_ref)
    hs_ref[...] = jnp.dot(x_ref[...], w_ref[...],
                          preferred_element_type=jnp.float32) * dinv


_tc1 = pl.pallas_call(
    _tc1_body,
    out_shape=jax.ShapeDtypeStruct((N, D), jnp.float32),
)


def _tc_comb_body(degp_t_ref, p_ref, hs_ref, b_ref, w_ref, z_ref, nxt_ref):
    dinv = _dinv(degp_t_ref)
    agg = p_ref[0] + p_ref[1] - hs_ref[...]
    z = jnp.maximum(agg * dinv + b_ref[...], 0.0)
    z_ref[...] = z
    nxt_ref[...] = jnp.dot(z, w_ref[...],
                           preferred_element_type=jnp.float32) * dinv


_tc_comb = pl.pallas_call(
    _tc_comb_body,
    out_shape=(jax.ShapeDtypeStruct((N, D), jnp.float32),
               jax.ShapeDtypeStruct((N, D), jnp.float32)),
)


def _tc_fin_body(z_ref, wlin_ref, blin_ref, out_ref):
    out_ref[...] = jnp.dot(z_ref[...], wlin_ref[...],
                           preferred_element_type=jnp.float32) + blin_ref[...]


_tc_fin = pl.pallas_call(
    _tc_fin_body,
    out_shape=jax.ShapeDtypeStruct((N, 1), jnp.float32),
)


def kernel(x, edge_index, batch, edge_attr, W1, b1, W2, b2, Wlin, blin):
    del batch, edge_attr
    edges_i32 = edge_index.astype(jnp.int32)
    edges = edges_i32.reshape(2, NW, NCHUNK, CH)
    eflat = edges_i32.reshape(2 * E)
    ones = jnp.ones((N,), jnp.float32)

    degp = _deg_sc(edges, ones)
    degp_t = degp.T

    hs1 = _tc1(degp_t, x, W1)

    p1 = _edge_sc(hs1, edges, eflat)
    z1, hs2 = _tc_comb(degp_t, p1, hs1, b1.reshape(1, D), W2)
    del z1
    p2 = _edge_sc(hs2, edges, eflat)
    z2, _unused = _tc_comb(degp_t, p2, hs2, b2.reshape(1, D), W2)
    out = _tc_fin(z2, Wlin, blin.reshape(1, 1))
    return out


# speedup vs baseline: 27.0411x; 1.0402x over previous
import functools

import jax
import jax.numpy as jnp
from jax import lax
from jax.experimental import pallas as pl
from jax.experimental.pallas import tpu as pltpu
from jax.experimental.pallas import tpu_sc as plsc

N = 10000
E = 320000
D = 128

NC = 2
NS = 16
NW = NC * NS
EPW = E // NW
CH = 80
NCHUNK = EPW // CH
RPT = 624
RREM = N - RPT * NS

_MESH = plsc.VectorSubcoreMesh(core_axis_name="c", subcore_axis_name="s")


NB = 2
NWAVE = NCHUNK // NB
NTAIL = NCHUNK - NWAVE * NB


@functools.partial(
    pl.kernel,
    mesh=_MESH,
    out_type=jax.ShapeDtypeStruct((NC, N), jnp.float32),
    scratch_types=[
        pltpu.VMEM((NCHUNK, CH), jnp.int32),
        pltpu.VMEM((CH,), jnp.float32),
        pltpu.VMEM_SHARED((N,), jnp.float32),
        [pltpu.SemaphoreType.DMA] * NB,
    ],
)
def _deg_sc(edge_hbm, ones_hbm, out_hbm, didx, ones_v, acc, sems):
    c = lax.axis_index("c")
    s = lax.axis_index("s")

    @pl.when(s == 0)
    def _():
        pltpu.sync_copy(ones_hbm, acc)

    pltpu.sync_copy(ones_hbm.at[pl.ds(0, CH)], ones_v)
    w = c * NS + s
    pltpu.sync_copy(edge_hbm.at[1, w], didx)
    plsc.subcore_barrier()

    def body(j, carry):
        ds = [
            pltpu.async_copy(ones_v, acc.at[didx.at[j * NB + b]], sems[b],
                             add=True)
            for b in range(NB)
        ]
        for d in ds:
            d.wait()
        return carry

    lax.fori_loop(0, NWAVE, body, 0)
    for t in range(NTAIL):
        pltpu.sync_copy(ones_v, acc.at[didx.at[NWAVE * NB + t]], add=True)
    plsc.subcore_barrier()

    @pl.when(s == 0)
    def _():
        pltpu.sync_copy(acc, out_hbm.at[c])


@functools.partial(
    pl.kernel,
    mesh=_MESH,
    out_type=jax.ShapeDtypeStruct((NC, N, D), jnp.float32),
    scratch_types=[
        pltpu.VMEM((4, CH), jnp.int32),
        pltpu.VMEM((4, CH), jnp.int32),
        pltpu.VMEM((4, CH, D), jnp.float32),
        pltpu.VMEM_SHARED((N, D), jnp.float32),
        pltpu.SemaphoreType.DMA((4,)),
        pltpu.SemaphoreType.DMA((4,)),
        pltpu.SemaphoreType.DMA((4,)),
    ],
)
def _edge_sc(hs_hbm, edge_hbm, eflat_hbm, out_hbm, sidx, didx, rows, acc,
             isem, gsem, ssem):
    c = lax.axis_index("c")
    s = lax.axis_index("s")

    r0 = s * RPT
    pltpu.sync_copy(hs_hbm.at[pl.ds(r0, RPT)], acc.at[pl.ds(r0, RPT)])

    @pl.when(s == NS - 1)
    def _():
        pltpu.sync_copy(hs_hbm.at[pl.ds(RPT * NS, RREM)],
                        acc.at[pl.ds(RPT * NS, RREM)])

    w = c * NS + s
    sb = w * EPW
    db = E + w * EPW
    plsc.subcore_barrier()

    def iload(t, j):
        pltpu.async_copy(eflat_hbm.at[pl.ds(sb + j * CH, CH)], sidx.at[t],
                         isem.at[t])
        pltpu.async_copy(eflat_hbm.at[pl.ds(db + j * CH, CH)], didx.at[t],
                         isem.at[t])
        pltpu.make_async_copy(eflat_hbm.at[pl.ds(0, CH)], sidx.at[t],
                              isem.at[t]).wait()
        pltpu.make_async_copy(eflat_hbm.at[pl.ds(0, CH)], didx.at[t],
                              isem.at[t]).wait()

    def gather(t):
        pltpu.async_copy(hs_hbm.at[sidx.at[t]], rows.at[t], gsem.at[t])

    def gwait(t):
        pltpu.make_async_copy(hs_hbm.at[pl.ds(0, CH)], rows.at[t],
                              gsem.at[t]).wait()

    def scatter(t):
        pltpu.async_copy(rows.at[t], acc.at[didx.at[t]], ssem.at[t], add=True)

    def swait(t):
        pltpu.make_async_copy(hs_hbm.at[pl.ds(0, CH)], rows.at[t],
                              ssem.at[t]).wait()

    def pload(t0, p):
        for b in range(2):
            iload(t0 + b, 2 * p + b)

    def pgather(t0):
        gather(t0)
        gather(t0 + 1)

    def pgwait(t0):
        gwait(t0)
        gwait(t0 + 1)

    def pscatter(t0):
        scatter(t0)
        scatter(t0 + 1)

    def pswait(t0):
        swait(t0)
        swait(t0 + 1)

    NPAIR = NCHUNK // 2
    K = NPAIR // 2

    def pload_clamped(t0, p):
        c0 = jnp.minimum(2 * p, NCHUNK - 1)
        c1 = jnp.minimum(2 * p + 1, NCHUNK - 1)
        iload(t0, c0)
        iload(t0 + 1, c1)

    iload(0, 0)
    iload(1, 1)
    pgather(0)

    def body(k, carry):
        pgwait(0)
        pscatter(0)
        @pl.when(k > 0)
        def _():
            pswait(2)

        pload(2, 2 * k + 1)
        pgather(2)
        pswait(0)
        pload_clamped(0, 2 * k + 2)
        pgather(0)
        pgwait(2)
        pscatter(2)
        return carry

    lax.fori_loop(0, K, body, 0)
    pswait(2)
    pgwait(0)
    scatter(0)
    swait(0)
    plsc.subcore_barrier()
    pltpu.sync_copy(acc.at[pl.ds(r0, RPT)], out_hbm.at[c, pl.ds(r0, RPT)])

    @pl.when(s == NS - 1)
    def _():
        pltpu.sync_copy(acc.at[pl.ds(RPT * NS, RREM)],
                        out_hbm.at[c, pl.ds(RPT * NS, RREM)])


def _dinv(degp_t_ref):
    deg = degp_t_ref[:, 0:1] + degp_t_ref[:, 1:2] - 1.0
    return lax.rsqrt(deg)


def _tc1_body(degp_t_ref, x_ref, w_ref, hs_ref):
    dinv = _dinv(degp_t_ref)
    hs_ref[...] = jnp.dot(x_ref[...], w_ref[...],
                          preferred_element_type=jnp.float32) * dinv


_tc1 = pl.pallas_call(
    _tc1_body,
    out_shape=jax.ShapeDtypeStruct((N, D), jnp.float32),
)


def _tc_comb_body(degp_t_ref, p_ref, hs_ref, b_ref, w_ref, z_ref, nxt_ref):
    dinv = _dinv(degp_t_ref)
    agg = p_ref[0] + p_ref[1] - hs_ref[...]
    z = jnp.maximum(agg * dinv + b_ref[...], 0.0)
    z_ref[...] = z
    nxt_ref[...] = jnp.dot(z, w_ref[...],
                           preferred_element_type=jnp.float32) * dinv


_tc_comb = pl.pallas_call(
    _tc_comb_body,
    out_shape=(jax.ShapeDtypeStruct((N, D), jnp.float32),
               jax.ShapeDtypeStruct((N, D), jnp.float32)),
)


def _tc_fin_body(z_ref, wlin_ref, blin_ref, out_ref):
    out_ref[...] = jnp.dot(z_ref[...], wlin_ref[...],
                           preferred_element_type=jnp.float32) + blin_ref[...]


_tc_fin = pl.pallas_call(
    _tc_fin_body,
    out_shape=jax.ShapeDtypeStruct((N, 1), jnp.float32),
)


def kernel(x, edge_index, batch, edge_attr, W1, b1, W2, b2, Wlin, blin):
    del batch, edge_attr
    edges_i32 = edge_index.astype(jnp.int32)
    edges = edges_i32.reshape(2, NW, NCHUNK, CH)
    eflat = edges_i32.reshape(2 * E)
    ones = jnp.ones((N,), jnp.float32)

    degp = _deg_sc(edges, ones)
    degp_t = degp.T

    hs1 = _tc1(degp_t, x, W1)

    p1 = _edge_sc(hs1, edges, eflat)
    z1, hs2 = _tc_comb(degp_t, p1, hs1, b1.reshape(1, D), W2)
    del z1
    p2 = _edge_sc(hs2, edges, eflat)
    z2, _unused = _tc_comb(degp_t, p2, hs2, b2.reshape(1, D), W2)
    out = _tc_fin(z2, Wlin, blin.reshape(1, 1))
    return out
